# baseline (speedup 1.0000x reference)
import functools

import jax
import jax.numpy as jnp
from jax import lax
from jax.experimental import pallas as pl
from jax.experimental.pallas import tpu as pltpu
from jax.experimental.pallas import tpu_sc as plsc

_LANES = 16


@functools.cache
def _build_sc_gather(b, l, l_pad, vocab, d):
    info = plsc.get_sparse_core_info()
    nc, ns = info.num_cores, info.num_subcores
    nw = nc * ns
    assert b % nw == 0
    rows_per_w = b // nw
    assert rows_per_w % 2 == 0 and rows_per_w >= 4 and rows_per_w % 8 == 0
    assert l <= 128 and l_pad % 8 == 0 and l_pad >= _LANES
    l_main = l - l % 8

    copy_offs = list(range(0, l_pad - _LANES + 1, _LANES))
    if copy_offs[-1] + _LANES < l_pad:
        copy_offs.append(l_pad - _LANES)

    mesh = plsc.VectorSubcoreMesh(core_axis_name="c", subcore_axis_name="s")

    @functools.partial(
        pl.kernel,
        mesh=mesh,
        out_type=[
            jax.ShapeDtypeStruct((b, l, d), jnp.float32),
            jax.ShapeDtypeStruct((b, 8, d), jnp.float32),
        ],
        scratch_types=[
            pltpu.VMEM((rows_per_w, l_pad), jnp.int32),
            pltpu.VMEM((l_pad,), jnp.int32),
            pltpu.VMEM((l_pad,), jnp.int32),
            pltpu.VMEM((l, d), jnp.float32),
            pltpu.VMEM((l, d), jnp.float32),
            pltpu.VMEM((8, d), jnp.float32),
            pltpu.VMEM((8, d), jnp.float32),
            pltpu.SemaphoreType.DMA,
            pltpu.SemaphoreType.DMA,
            pltpu.SemaphoreType.DMA,
            pltpu.SemaphoreType.DMA,
        ],
    )
    def sc_gather(ids_hbm, table_hbm, emb_out, tail_out,
                  idx_v, tmp_a, tmp_b, buf_a, buf_b, tl_a, tl_b,
                  gsa, gsb, ssa, ssb):
        wid = lax.axis_index("s") * nc + lax.axis_index("c")
        base = wid * rows_per_w

        pltpu.sync_copy(ids_hbm.at[wid], idx_v)

        def build(j, tmp):
            for off in copy_offs:
                tmp[pl.ds(off, _LANES)] = idx_v[j, pl.ds(off, _LANES)]

        def gathers(tmp, buf, tl, sem):
            g1 = pltpu.make_async_copy(
                table_hbm.at[tmp.at[pl.ds(0, l)]], buf, sem)
            g2 = pltpu.make_async_copy(
                table_hbm.at[tmp.at[pl.ds(l_main, 8)]], tl, sem)
            return g1, g2

        def scatters(j, buf, tl, sem):
            s1 = pltpu.make_async_copy(buf, emb_out.at[base + j], sem)
            s2 = pltpu.make_async_copy(tl, tail_out.at[base + j], sem)
            return s1, s2

        def start(ops):
            for op in ops:
                op.start()

        def wait(ops):
            for op in ops:
                op.wait()

        build(0, tmp_a)
        start(gathers(tmp_a, buf_a, tl_a, gsa))
        wait(gathers(tmp_a, buf_a, tl_a, gsa))
        build(1, tmp_b)
        start(gathers(tmp_b, buf_b, tl_b, gsb))
        start(scatters(0, buf_a, tl_a, ssa))

        def pipe(i, carry):
            c = 2 * i + 1
            wait(gathers(tmp_b, buf_b, tl_b, gsb))
            wait(scatters(c - 1, buf_a, tl_a, ssa))
            build(c + 1, tmp_a)
            start(gathers(tmp_a, buf_a, tl_a, gsa))
            start(scatters(c, buf_b, tl_b, ssb))
            wait(gathers(tmp_a, buf_a, tl_a, gsa))
            wait(scatters(c, buf_b, tl_b, ssb))
            build(c + 2, tmp_b)
            start(gathers(tmp_b, buf_b, tl_b, gsb))
            start(scatters(c + 1, buf_a, tl_a, ssa))
            return carry

        lax.fori_loop(0, rows_per_w // 2 - 1, pipe, 0)

        last = rows_per_w - 1
        wait(gathers(tmp_b, buf_b, tl_b, gsb))
        wait(scatters(last - 1, buf_a, tl_a, ssa))
        start(scatters(last, buf_b, tl_b, ssb))
        wait(scatters(last, buf_b, tl_b, ssb))

    return sc_gather


@functools.cache
def _build_patch_ts(b, l, d):
    inv = 1.0 / float(l - 1)
    l_main = l - l % 8
    tail = l - l_main

    def body(emb_in, tail_in, emb_o, ts_o, vbuf, sem):
        del emb_in
        pos = lax.broadcasted_iota(jnp.int32, (b, l), 1)
        ts_o[...] = pos.astype(jnp.float32) * inv
        pltpu.make_async_copy(
            tail_in.at[:, pl.ds(0, tail), :], vbuf, sem).start()
        pltpu.make_async_copy(
            tail_in.at[:, pl.ds(0, tail), :], vbuf, sem).wait()
        pltpu.make_async_copy(
            vbuf, emb_o.at[:, pl.ds(l_main, tail), :], sem).start()
        pltpu.make_async_copy(
            vbuf, emb_o.at[:, pl.ds(l_main, tail), :], sem).wait()

    return pl.pallas_call(
        body,
        in_specs=[
            pl.BlockSpec(memory_space=pl.ANY),
            pl.BlockSpec(memory_space=pl.ANY),
        ],
        out_specs=[
            pl.BlockSpec(memory_space=pl.ANY),
            pl.BlockSpec((b, l), lambda: (0, 0)),
        ],
        out_shape=[
            jax.ShapeDtypeStruct((b, l, d), jnp.float32),
            jax.ShapeDtypeStruct((b, l), jnp.float32),
        ],
        scratch_shapes=[
            pltpu.VMEM((b, tail, d), jnp.float32),
            pltpu.SemaphoreType.DMA,
        ],
        input_output_aliases={0: 0},
    )


def kernel(input_ids, table):
    b, l = input_ids.shape
    vocab, d = table.shape
    nw = 32
    l_pad = max((l + 7) // 8 * 8, _LANES)
    ids = input_ids.astype(jnp.int32)
    ids_pad = jnp.pad(ids, ((0, 0), (0, l_pad - l))).reshape(nw, b // nw, l_pad)
    emb0, tail = _build_sc_gather(b, l, l_pad, vocab, d)(ids_pad, table)
    emb, ts = _build_patch_ts(b, l, d)(emb0, tail)
    return emb, ts
